# Initial kernel scaffold; baseline (speedup 1.0000x reference)
import jax
import jax.numpy as jnp
from jax.experimental import pallas as pl


def kernel(batch_basket, table):
    raise NotImplementedError("write your pallas kernel here")


# speedup vs baseline: 12.4486x; 12.4486x over previous
import functools

import jax
import jax.numpy as jnp
from jax import lax
from jax.experimental import pallas as pl
from jax.experimental.pallas import tpu as pltpu
from jax.experimental.pallas import tpu_sc as plsc

HIDDEN = 64
K = 20
NC, NS, L = 2, 16, 16
NW = NC * NS
TOTAL_BASKETS = 1024 * 50
B_PER_W = TOTAL_BASKETS // NW
CHUNK_B = 32
N_CHUNKS = B_PER_W // CHUNK_B
IDX_PER_CHUNK = CHUNK_B * K
N_GATHERS = IDX_PER_CHUNK // 128
NVREG = HIDDEN // L


def _body(idx_hbm, table_hbm, out_hbm, idx_v, rows_v, out_v,
          gsem0, gsem1, osem0, osem1):
    wid = lax.axis_index("s") * NC + lax.axis_index("c")

    def fire_gathers(slot, sem):
        for j in range(N_GATHERS):
            pltpu.async_copy(
                table_hbm.at[idx_v.at[slot, j]],
                rows_v.at[slot, pl.ds(j * 128, 128)],
                sem,
            )

    def wait_gathers(slot, sem):
        for j in range(N_GATHERS):
            pltpu.make_async_copy(
                table_hbm.at[idx_v.at[slot, j]],
                rows_v.at[slot, pl.ds(j * 128, 128)],
                sem,
            ).wait()

    def load_idx(g, slot):
        pltpu.sync_copy(idx_hbm.at[wid, g], idx_v.at[slot])

    def compute_chunk(g, slot):
        def basket(c, _):
            base = c * K
            for j in range(NVREG):
                acc = rows_v[slot, base, pl.ds(j * L, L)]
                for k in range(1, K):
                    acc = acc + rows_v[slot, base + k, pl.ds(j * L, L)]
                out_v[slot, c, pl.ds(j * L, L)] = acc * jnp.float32(1.0 / K)
            return _
        lax.fori_loop(0, CHUNK_B, basket, None)
        pltpu.async_copy(
            out_v.at[slot],
            out_hbm.at[pl.ds(wid * B_PER_W + g * CHUNK_B, CHUNK_B)],
            osems[slot],
        )

    def wait_out(slot):
        pltpu.make_async_copy(
            out_v.at[slot],
            out_hbm.at[pl.ds(wid * B_PER_W, CHUNK_B)],
            osems[slot],
        ).wait()

    gsems = (gsem0, gsem1)
    osems = (osem0, osem1)

    load_idx(0, 0)
    fire_gathers(0, gsem0)

    @pl.loop(0, N_CHUNKS, step=2)
    def _chunks(g0):
        for b in range(2):
            g = g0 + b
            nxt = 1 - b
            if b == 0:
                load_idx(g + 1, nxt)
                fire_gathers(nxt, gsems[nxt])
            else:
                @pl.when(g0 < N_CHUNKS - 2)
                def _():
                    load_idx(g + 1, nxt)
                    fire_gathers(nxt, gsems[nxt])
            wait_gathers(b, gsems[b])
            @pl.when(g >= 2)
            def _():
                wait_out(b)
            compute_chunk(g, b)

    wait_out(0)
    wait_out(1)


@jax.jit
def _pooled(idx, table):
    mesh = plsc.VectorSubcoreMesh(
        core_axis_name="c", subcore_axis_name="s",
        num_cores=NC, num_subcores=NS,
    )
    run = functools.partial(
        pl.kernel,
        out_type=jax.ShapeDtypeStruct((TOTAL_BASKETS, HIDDEN), jnp.float32),
        mesh=mesh,
        compiler_params=pltpu.CompilerParams(use_tc_tiling_on_sc=False),
        scratch_types=[
            pltpu.VMEM((2, N_GATHERS, 128), jnp.int32),
            pltpu.VMEM((2, IDX_PER_CHUNK, HIDDEN), jnp.float32),
            pltpu.VMEM((2, CHUNK_B, HIDDEN), jnp.float32),
            pltpu.SemaphoreType.DMA,
            pltpu.SemaphoreType.DMA,
            pltpu.SemaphoreType.DMA,
            pltpu.SemaphoreType.DMA,
        ],
    )(_body)
    return run(idx, table)


def kernel(batch_basket, table):
    idx = batch_basket.reshape(NW, N_CHUNKS, N_GATHERS, 128)
    out = _pooled(idx, table)
    return out.reshape(1024, 50, HIDDEN)


# speedup vs baseline: 13.5039x; 1.0848x over previous
import functools

import jax
import jax.numpy as jnp
from jax import lax
from jax.experimental import pallas as pl
from jax.experimental.pallas import tpu as pltpu
from jax.experimental.pallas import tpu_sc as plsc

HIDDEN = 64
K = 20
NC, NS, L = 2, 16, 16
NW = NC * NS
TOTAL_BASKETS = 1024 * 50
B_PER_W = TOTAL_BASKETS // NW
CHUNK_B = 32
N_CHUNKS = B_PER_W // CHUNK_B
IDX_PER_CHUNK = CHUNK_B * K
N_GATHERS = IDX_PER_CHUNK // 128
NVREG = HIDDEN // L


def _body(idx_hbm, table_hbm, out_hbm, idx_v, rows_v, out_v,
          gsem0, gsem1, osem0, osem1):
    wid = lax.axis_index("s") * NC + lax.axis_index("c")

    def fire_gathers(g, slot, sem):
        for j in range(N_GATHERS):
            pltpu.async_copy(
                table_hbm.at[idx_v.at[g * N_GATHERS + j]],
                rows_v.at[slot, pl.ds(j * 128, 128)],
                sem,
            )

    def wait_gathers(slot, sem):
        for j in range(N_GATHERS):
            pltpu.make_async_copy(
                table_hbm.at[idx_v.at[j]],
                rows_v.at[slot, pl.ds(j * 128, 128)],
                sem,
            ).wait()

    def compute_chunk(g, slot):
        @pl.loop(0, CHUNK_B, unroll=2)
        def basket(c):
            base = c * K
            for j in range(NVREG):
                acc = rows_v[slot, base, pl.ds(j * L, L)]
                for k in range(1, K):
                    acc = acc + rows_v[slot, base + k, pl.ds(j * L, L)]
                out_v[slot, c, pl.ds(j * L, L)] = acc * jnp.float32(1.0 / K)
        pltpu.async_copy(
            out_v.at[slot],
            out_hbm.at[pl.ds(wid * B_PER_W + g * CHUNK_B, CHUNK_B)],
            osems[slot],
        )

    def wait_out(slot):
        pltpu.make_async_copy(
            out_v.at[slot],
            out_hbm.at[pl.ds(wid * B_PER_W, CHUNK_B)],
            osems[slot],
        ).wait()

    gsems = (gsem0, gsem1)
    osems = (osem0, osem1)

    pltpu.sync_copy(idx_hbm.at[wid], idx_v)
    fire_gathers(0, 0, gsem0)

    @pl.loop(0, N_CHUNKS, step=2)
    def _chunks(g0):
        for b in range(2):
            g = g0 + b
            nxt = 1 - b
            if b == 0:
                fire_gathers(g + 1, nxt, gsems[nxt])
            else:
                @pl.when(g0 < N_CHUNKS - 2)
                def _():
                    fire_gathers(g + 1, nxt, gsems[nxt])
            wait_gathers(b, gsems[b])
            @pl.when(g >= 2)
            def _():
                wait_out(b)
            compute_chunk(g, b)

    wait_out(0)
    wait_out(1)


@jax.jit
def _pooled(idx, table):
    mesh = plsc.VectorSubcoreMesh(
        core_axis_name="c", subcore_axis_name="s",
        num_cores=NC, num_subcores=NS,
    )
    run = functools.partial(
        pl.kernel,
        out_type=jax.ShapeDtypeStruct((TOTAL_BASKETS, HIDDEN), jnp.float32),
        mesh=mesh,
        compiler_params=pltpu.CompilerParams(use_tc_tiling_on_sc=False),
        scratch_types=[
            pltpu.VMEM((N_CHUNKS * N_GATHERS, 128), jnp.int32),
            pltpu.VMEM((2, IDX_PER_CHUNK, HIDDEN), jnp.float32),
            pltpu.VMEM((2, CHUNK_B, HIDDEN), jnp.float32),
            pltpu.SemaphoreType.DMA,
            pltpu.SemaphoreType.DMA,
            pltpu.SemaphoreType.DMA,
            pltpu.SemaphoreType.DMA,
        ],
    )(_body)
    return run(idx, table)


def kernel(batch_basket, table):
    idx = batch_basket.reshape(NW, N_CHUNKS * N_GATHERS, 128)
    out = _pooled(idx, table)
    return out.reshape(1024, 50, HIDDEN)


# speedup vs baseline: 15.8897x; 1.1767x over previous
import functools

import jax
import jax.numpy as jnp
from jax import lax
from jax.experimental import pallas as pl
from jax.experimental.pallas import tpu as pltpu
from jax.experimental.pallas import tpu_sc as plsc

HIDDEN = 64
K = 20
NC, NS, L = 2, 16, 16
NW = NC * NS
TOTAL_BASKETS = 1024 * 50
B_PER_W = TOTAL_BASKETS // NW
CHUNK_B = 32
N_CHUNKS = B_PER_W // CHUNK_B
IDX_PER_CHUNK = CHUNK_B * K
N_GATHERS = IDX_PER_CHUNK // 128
NVREG = HIDDEN // L


def _body(idx_hbm, table_hbm, out_hbm, idx_v, rows_v, out_v,
          gsem0, gsem1, osem0, osem1):
    wid = lax.axis_index("s") * NC + lax.axis_index("c")

    def fire_gathers(g, slot, sem):
        for j in range(N_GATHERS):
            pltpu.async_copy(
                table_hbm.at[idx_v.at[g * N_GATHERS + j]],
                rows_v.at[slot, pl.ds(j * 128, 128)],
                sem,
            )

    def wait_gathers(slot, sem):
        for j in range(N_GATHERS):
            pltpu.make_async_copy(
                table_hbm.at[idx_v.at[j]],
                rows_v.at[slot, pl.ds(j * 128, 128)],
                sem,
            ).wait()

    def compute_chunk(g, slot):
        @pl.loop(0, CHUNK_B, unroll=2)
        def basket(c):
            base = c * K
            for j in range(NVREG):
                vs = [rows_v[slot, base + k, pl.ds(j * L, L)] +
                      rows_v[slot, base + k + 1, pl.ds(j * L, L)]
                      for k in range(0, K, 2)]
                while len(vs) > 1:
                    nxt_vs = [vs[i] + vs[i + 1] for i in range(0, len(vs) - 1, 2)]
                    if len(vs) % 2:
                        nxt_vs.append(vs[-1])
                    vs = nxt_vs
                out_v[slot, c, pl.ds(j * L, L)] = vs[0] * jnp.float32(1.0 / K)
        pltpu.async_copy(
            out_v.at[slot],
            out_hbm.at[pl.ds(wid * B_PER_W + g * CHUNK_B, CHUNK_B)],
            osems[slot],
        )

    def wait_out(slot):
        pltpu.make_async_copy(
            out_v.at[slot],
            out_hbm.at[pl.ds(wid * B_PER_W, CHUNK_B)],
            osems[slot],
        ).wait()

    gsems = (gsem0, gsem1)
    osems = (osem0, osem1)

    pltpu.sync_copy(idx_hbm.at[wid], idx_v)
    fire_gathers(0, 0, gsem0)

    @pl.loop(0, N_CHUNKS, step=2)
    def _chunks(g0):
        for b in range(2):
            g = g0 + b
            nxt = 1 - b
            if b == 0:
                fire_gathers(g + 1, nxt, gsems[nxt])
            else:
                @pl.when(g0 < N_CHUNKS - 2)
                def _():
                    fire_gathers(g + 1, nxt, gsems[nxt])
            wait_gathers(b, gsems[b])
            @pl.when(g >= 2)
            def _():
                wait_out(b)
            compute_chunk(g, b)

    wait_out(0)
    wait_out(1)


@jax.jit
def _pooled(idx, table):
    mesh = plsc.VectorSubcoreMesh(
        core_axis_name="c", subcore_axis_name="s",
        num_cores=NC, num_subcores=NS,
    )
    run = functools.partial(
        pl.kernel,
        out_type=jax.ShapeDtypeStruct((TOTAL_BASKETS, HIDDEN), jnp.float32),
        mesh=mesh,
        compiler_params=pltpu.CompilerParams(use_tc_tiling_on_sc=False),
        scratch_types=[
            pltpu.VMEM((N_CHUNKS * N_GATHERS, 128), jnp.int32),
            pltpu.VMEM((2, IDX_PER_CHUNK, HIDDEN), jnp.float32),
            pltpu.VMEM((2, CHUNK_B, HIDDEN), jnp.float32),
            pltpu.SemaphoreType.DMA,
            pltpu.SemaphoreType.DMA,
            pltpu.SemaphoreType.DMA,
            pltpu.SemaphoreType.DMA,
        ],
    )(_body)
    return run(idx, table)


def kernel(batch_basket, table):
    idx = batch_basket.reshape(NW, N_CHUNKS * N_GATHERS, 128)
    out = _pooled(idx, table)
    return out.reshape(1024, 50, HIDDEN)
